# baseline (speedup 1.0000x reference)
import functools

import jax
import jax.numpy as jnp
from jax import lax
from jax.experimental import pallas as pl
from jax.experimental.pallas import tpu as pltpu
from jax.experimental.pallas import tpu_sc as plsc

_B = 4
_S = 4
_D = 2048

_mesh = plsc.ScalarSubcoreMesh(axis_name="c", num_cores=1)


@functools.partial(
    pl.kernel,
    mesh=_mesh,
    out_type=jax.ShapeDtypeStruct((_B, _S, _D), jnp.float32),
)
def _gather_head(x_hbm, out_hbm):
    pltpu.sync_copy(x_hbm.at[:, pl.ds(0, _S), 0], out_hbm)


def kernel(x):
    return _gather_head(x)


# speedup vs baseline: 1.2025x; 1.2025x over previous
import functools

import jax
import jax.numpy as jnp
from jax import lax
from jax.experimental import pallas as pl
from jax.experimental.pallas import tpu as pltpu
from jax.experimental.pallas import tpu_sc as plsc

_B = 4
_S = 4
_D = 2048
_NC = 1
_NS = 16
_NW = _NC * _NS
_CHUNK = (_B * _S * _D) // _NW
_PER_ROW = _D // _CHUNK

_mesh = plsc.ScalarSubcoreMesh(axis_name="c", num_cores=1)


@functools.partial(
    pl.kernel,
    mesh=_mesh,
    out_type=jax.ShapeDtypeStruct((_B, _S, _D), jnp.float32),
    scratch_types=[
        pltpu.VMEM_SHARED((_B, _S, _D), jnp.float32),
        pltpu.SemaphoreType.DMA,
        pltpu.SemaphoreType.DMA,
    ],
)
def _gather_head(x_hbm, out_hbm, stage, in_sem, out_sem):
    ins = [
        pltpu.make_async_copy(
            x_hbm.at[b, pl.ds(0, _S), 0], stage.at[b], in_sem
        )
        for b in range(_B)
    ]
    outs = [
        pltpu.make_async_copy(stage.at[b], out_hbm.at[b], out_sem)
        for b in range(_B)
    ]
    for c in ins:
        c.start()
    for b in range(_B):
        ins[b].wait()
        outs[b].start()
    for c in outs:
        c.wait()


def kernel(x):
    return _gather_head(x)
